# Initial kernel scaffold; baseline (speedup 1.0000x reference)
import jax
import jax.numpy as jnp
from jax.experimental import pallas as pl


def kernel(point_cloud, query_cloud, temperature):
    raise NotImplementedError("write your pallas kernel here")


# speedup vs baseline: 33.9801x; 33.9801x over previous
import functools

import jax
import jax.numpy as jnp
from jax.experimental import pallas as pl
from jax.experimental.pallas import tpu as pltpu

_K = 16
_MIN_SIGMA = 1e-4


def _soft_proj_block(p_ref, q_ref, sig_ref, out_ref, *, n, bm):
    p = p_ref[0]
    q = q_ref[0]
    inv_sigma = 1.0 / (sig_ref[0, 0] + 1e-8)

    p2 = jnp.sum(p * p, axis=0, keepdims=True)
    q2 = jnp.sum(q * q, axis=0)[:, None]
    inner = jax.lax.dot_general(
        q, p, (((0,), (0,)), ((), ())),
        preferred_element_type=jnp.float32)
    d2 = q2 - 2.0 * inner + p2

    work = d2
    tau = None
    for _ in range(_K):
        tau = jnp.min(work, axis=1, keepdims=True)
        work = jnp.where(work <= tau, jnp.inf, work)

    dmin = jnp.min(d2, axis=1, keepdims=True)
    e = jnp.where(d2 <= tau,
                  jnp.exp((dmin - d2) * inv_sigma), 0.0)
    denom = jnp.sum(e, axis=1)[None, :]
    num = jax.lax.dot_general(
        p, e, (((1,), (1,)), ((), ())),
        preferred_element_type=jnp.float32)
    out_ref[0] = num / denom


def kernel(point_cloud, query_cloud, temperature):
    b, c, n = point_cloud.shape
    _, _, m = query_cloud.shape
    bm = 256
    sigma = jnp.maximum(temperature * temperature, jnp.float32(_MIN_SIGMA))
    sigma = jnp.reshape(sigma, (1, 1)).astype(jnp.float32)

    grid = (b, m // bm)
    return pl.pallas_call(
        functools.partial(_soft_proj_block, n=n, bm=bm),
        grid=grid,
        in_specs=[
            pl.BlockSpec((1, c, n), lambda i, j: (i, 0, 0)),
            pl.BlockSpec((1, c, bm), lambda i, j: (i, 0, j)),
            pl.BlockSpec(memory_space=pltpu.SMEM),
        ],
        out_specs=pl.BlockSpec((1, c, bm), lambda i, j: (i, 0, j)),
        out_shape=jax.ShapeDtypeStruct((b, c, m), jnp.float32),
    )(point_cloud, query_cloud, sigma)


# speedup vs baseline: 36.4730x; 1.0734x over previous
import functools

import jax
import jax.numpy as jnp
from jax.experimental import pallas as pl
from jax.experimental.pallas import tpu as pltpu

_K = 16
_NG = 512
_GS = 16
_MIN_SIGMA = 1e-4
_BIG = 3.0e38


def _extract_min_onehots(vals, iota, steps):
    n = vals.shape[1]
    onehots = []
    for _ in range(steps):
        vmin = jnp.min(vals, axis=1, keepdims=True)
        idxm = jnp.where(vals == vmin, iota, n)
        jmin = jnp.min(idxm, axis=1, keepdims=True)
        oh = idxm == jmin
        onehots.append(oh)
        vals = jnp.where(oh, _BIG, vals)
    return onehots, vals


def _bf(x):
    return x.astype(jnp.bfloat16).astype(jnp.float32)


def _soft_proj_block(p_ref, q_ref, qt_ref, pg_ref, sig_ref, out_ref, *, n, bm):
    p = p_ref[0]
    q = q_ref[0]
    qt = qt_ref[0]
    pg = pg_ref[0]
    inv_sigma = 1.0 / (sig_ref[0, 0] + 1e-8)

    px = p[0:1, :]
    py = p[1:2, :]
    pz = p[2:3, :]
    p2 = (px * px + py * py) + pz * pz
    q2 = jnp.sum(qt * qt, axis=1, keepdims=True)
    inner = jax.lax.dot_general(
        q, p, (((0,), (0,)), ((), ())),
        preferred_element_type=jnp.float32)
    d2 = q2 - 2.0 * inner + p2

    g = d2[:, 0:_NG]
    for e in range(1, _GS):
        g = jnp.minimum(g, d2[:, e * _NG:(e + 1) * _NG])

    iota_g = jax.lax.broadcasted_iota(jnp.int32, (bm, _NG), 1)
    onehots, _ = _extract_min_onehots(g, iota_g, _K)
    ohcat = jnp.concatenate([oh.astype(jnp.float32) for oh in onehots],
                            axis=0)

    xcat = jax.lax.dot_general(
        ohcat, pg, (((1,), (0,)), ((), ())),
        preferred_element_type=jnp.float32,
        precision=jax.lax.Precision.HIGHEST)

    xs = []
    for c in range(3):
        cols = []
        for j in range(_K):
            cols.append(xcat[j * bm:(j + 1) * bm, c * 16:(c + 1) * 16])
        xs.append(jnp.concatenate(cols, axis=1))
    xx, xy, xz = xs

    qx = qt[:, 0:1]
    qy = qt[:, 1:2]
    qz = qt[:, 2:3]
    innerb = (_bf(qx) * _bf(xx) + _bf(qy) * _bf(xy)) + _bf(qz) * _bf(xz)
    p2x = (xx * xx + xy * xy) + xz * xz
    cd = q2 - 2.0 * innerb + p2x

    iota_c = jax.lax.broadcasted_iota(jnp.int32, (bm, _K * _GS), 1)
    sel_onehots, _ = _extract_min_onehots(cd, iota_c, _K)
    sel = sel_onehots[0]
    for oh in sel_onehots[1:]:
        sel = jnp.logical_or(sel, oh)

    dx = xx - qx
    dy = xy - qy
    dz = xz - qz
    ed = (dx * dx + dy * dy) + dz * dz

    dmin = jnp.min(jnp.where(sel, ed, _BIG), axis=1, keepdims=True)
    w = jnp.where(sel, jnp.exp((dmin - ed) * inv_sigma), 0.0)
    denom = jnp.sum(w, axis=1, keepdims=True)
    ox = jnp.sum(w * xx, axis=1, keepdims=True) / denom
    oy = jnp.sum(w * xy, axis=1, keepdims=True) / denom
    oz = jnp.sum(w * xz, axis=1, keepdims=True) / denom
    out_ref[0] = jnp.concatenate([ox, oy, oz], axis=1)


def kernel(point_cloud, query_cloud, temperature):
    b, c, n = point_cloud.shape
    _, _, m = query_cloud.shape
    bm = 256
    sigma = jnp.maximum(temperature * temperature, jnp.float32(_MIN_SIGMA))
    sigma = jnp.reshape(sigma, (1, 1)).astype(jnp.float32)

    pg = jnp.transpose(
        jnp.reshape(point_cloud, (b, c, _GS, _NG)), (0, 3, 1, 2)
    ).reshape(b, _NG, c * _GS)
    qt = jnp.transpose(query_cloud, (0, 2, 1))

    grid = (b, m // bm)
    out = pl.pallas_call(
        functools.partial(_soft_proj_block, n=n, bm=bm),
        grid=grid,
        in_specs=[
            pl.BlockSpec((1, c, n), lambda i, j: (i, 0, 0)),
            pl.BlockSpec((1, c, bm), lambda i, j: (i, 0, j)),
            pl.BlockSpec((1, bm, c), lambda i, j: (i, j, 0)),
            pl.BlockSpec((1, _NG, c * _GS), lambda i, j: (i, 0, 0)),
            pl.BlockSpec(memory_space=pltpu.SMEM),
        ],
        out_specs=pl.BlockSpec((1, bm, c), lambda i, j: (i, j, 0)),
        out_shape=jax.ShapeDtypeStruct((b, m, c), jnp.float32),
    )(point_cloud, query_cloud, qt, pg, sigma)
    return jnp.transpose(out, (0, 2, 1))


# speedup vs baseline: 42.1936x; 1.1568x over previous
import functools

import jax
import jax.numpy as jnp
from jax.experimental import pallas as pl
from jax.experimental.pallas import tpu as pltpu

_K = 16
_NG = 512
_GS = 16
_MIN_SIGMA = 1e-4
_BIG = 3.0e38
_IMIN = -2**31


def _total_order_keys(vals, nbits_idx, inv_iota):
    vi = jax.lax.bitcast_convert_type(vals, jnp.int32)
    ki = jnp.where(vi < 0, vi ^ 0x7FFFFFFF, vi)
    mask = jnp.int32(-(1 << nbits_idx))
    return jnp.bitwise_or(jnp.bitwise_and(ki, mask), inv_iota)


def _extract_max_onehots(keys, steps):
    onehots = []
    for _ in range(steps):
        kmax = jnp.max(keys, axis=1, keepdims=True)
        oh = keys == kmax
        onehots.append(oh)
        keys = jnp.where(oh, _IMIN, keys)
    return onehots


def _bf(x):
    return x.astype(jnp.bfloat16).astype(jnp.float32)


def _soft_proj_block(p_ref, q_ref, qt_ref, pg_ref, sig_ref, out_ref, *, n, bm):
    p = p_ref[0]
    q = q_ref[0]
    qt = qt_ref[0]
    pg = pg_ref[0]
    inv_sigma = 1.0 / (sig_ref[0, 0] + 1e-8)

    px = p[0:1, :]
    py = p[1:2, :]
    pz = p[2:3, :]
    hp = 0.5 * ((px * px + py * py) + pz * pz)
    inner = jax.lax.dot_general(
        q, p, (((0,), (0,)), ((), ())),
        preferred_element_type=jnp.float32)

    g = inner[:, 0:_NG] - hp[:, 0:_NG]
    for e in range(1, _GS):
        s = slice(e * _NG, (e + 1) * _NG)
        g = jnp.maximum(g, inner[:, s] - hp[:, s])

    iota_g = jax.lax.broadcasted_iota(jnp.int32, (bm, _NG), 1)
    keys = _total_order_keys(g, 9, (_NG - 1) - iota_g)
    onehots = _extract_max_onehots(keys, _K)
    ohcat = jnp.concatenate([oh.astype(jnp.float32) for oh in onehots],
                            axis=0)

    xcat = jax.lax.dot_general(
        ohcat, pg, (((1,), (0,)), ((), ())),
        preferred_element_type=jnp.float32,
        precision=jax.lax.Precision.HIGHEST)

    xs = []
    for c in range(3):
        cols = []
        for j in range(_K):
            cols.append(xcat[j * bm:(j + 1) * bm, c * 16:(c + 1) * 16])
        xs.append(jnp.concatenate(cols, axis=1))
    xx, xy, xz = xs

    qx = qt[:, 0:1]
    qy = qt[:, 1:2]
    qz = qt[:, 2:3]
    innerb = (_bf(qx) * _bf(xx) + _bf(qy) * _bf(xy)) + _bf(qz) * _bf(xz)
    hpx = 0.5 * ((xx * xx + xy * xy) + xz * xz)
    zc = innerb - hpx

    iota_c = jax.lax.broadcasted_iota(jnp.int32, (bm, _K * _GS), 1)
    ckeys = _total_order_keys(zc, 8, (_K * _GS - 1) - iota_c)
    sel_onehots = _extract_max_onehots(ckeys, _K)
    sel = sel_onehots[0]
    for oh in sel_onehots[1:]:
        sel = jnp.logical_or(sel, oh)

    dx = xx - qx
    dy = xy - qy
    dz = xz - qz
    ed = (dx * dx + dy * dy) + dz * dz

    dmin = jnp.min(jnp.where(sel, ed, _BIG), axis=1, keepdims=True)
    w = jnp.where(sel, jnp.exp((dmin - ed) * inv_sigma), 0.0)
    denom = jnp.sum(w, axis=1, keepdims=True)
    ox = jnp.sum(w * xx, axis=1, keepdims=True) / denom
    oy = jnp.sum(w * xy, axis=1, keepdims=True) / denom
    oz = jnp.sum(w * xz, axis=1, keepdims=True) / denom
    out_ref[0] = jnp.concatenate([ox, oy, oz], axis=1)


def kernel(point_cloud, query_cloud, temperature):
    b, c, n = point_cloud.shape
    _, _, m = query_cloud.shape
    bm = 256
    sigma = jnp.maximum(temperature * temperature, jnp.float32(_MIN_SIGMA))
    sigma = jnp.reshape(sigma, (1, 1)).astype(jnp.float32)

    pg = jnp.transpose(
        jnp.reshape(point_cloud, (b, c, _GS, _NG)), (0, 3, 1, 2)
    ).reshape(b, _NG, c * _GS)
    qt = jnp.transpose(query_cloud, (0, 2, 1))

    grid = (b, m // bm)
    out = pl.pallas_call(
        functools.partial(_soft_proj_block, n=n, bm=bm),
        grid=grid,
        in_specs=[
            pl.BlockSpec((1, c, n), lambda i, j: (i, 0, 0)),
            pl.BlockSpec((1, c, bm), lambda i, j: (i, 0, j)),
            pl.BlockSpec((1, bm, c), lambda i, j: (i, j, 0)),
            pl.BlockSpec((1, _NG, c * _GS), lambda i, j: (i, 0, 0)),
            pl.BlockSpec(memory_space=pltpu.SMEM),
        ],
        out_specs=pl.BlockSpec((1, bm, c), lambda i, j: (i, j, 0)),
        out_shape=jax.ShapeDtypeStruct((b, m, c), jnp.float32),
    )(point_cloud, query_cloud, qt, pg, sigma)
    return jnp.transpose(out, (0, 2, 1))


# speedup vs baseline: 45.2020x; 1.0713x over previous
import functools

import jax
import jax.numpy as jnp
from jax.experimental import pallas as pl
from jax.experimental.pallas import tpu as pltpu

_K = 16
_NG = 512
_GS = 16
_MIN_SIGMA = 1e-4
_BIG = 3.0e38
_IMIN = -2**31


def _total_order_keys(vals, nbits_idx, inv_iota):
    vi = jax.lax.bitcast_convert_type(vals, jnp.int32)
    ki = jnp.where(vi < 0, vi ^ 0x7FFFFFFF, vi)
    mask = jnp.int32(-(1 << nbits_idx))
    return jnp.bitwise_or(jnp.bitwise_and(ki, mask), inv_iota)


def _extract_max_onehots(keys, steps):
    onehots = []
    for _ in range(steps):
        kmax = jnp.max(keys, axis=1, keepdims=True)
        oh = keys == kmax
        onehots.append(oh)
        keys = jnp.where(oh, _IMIN, keys)
    return onehots


def _bf(x):
    return x.astype(jnp.bfloat16).astype(jnp.float32)


def _soft_proj_block(p_ref, q_ref, qt_ref, pg_ref, sig_ref, out_ref, *, n, bm):
    p = p_ref[0]
    q = q_ref[0]
    qt = qt_ref[0]
    pg = pg_ref[0]
    inv_sigma = 1.0 / (sig_ref[0, 0] + 1e-8)

    px = p[0:1, :]
    py = p[1:2, :]
    pz = p[2:3, :]
    hp = 0.5 * ((px * px + py * py) + pz * pz)
    inner = jax.lax.dot_general(
        q, p, (((0,), (0,)), ((), ())),
        preferred_element_type=jnp.float32)

    g = inner[:, 0:_NG] - hp[:, 0:_NG]
    for e in range(1, _GS):
        s = slice(e * _NG, (e + 1) * _NG)
        g = jnp.maximum(g, inner[:, s] - hp[:, s])

    iota_g = jax.lax.broadcasted_iota(jnp.int32, (bm, _NG), 1)
    keys = _total_order_keys(g, 9, (_NG - 1) - iota_g)
    onehots = _extract_max_onehots(keys, _K)
    ohcat = jnp.concatenate([oh.astype(jnp.bfloat16) for oh in onehots],
                            axis=0)

    xparts = jax.lax.dot_general(
        ohcat, pg, (((1,), (0,)), ((), ())),
        preferred_element_type=jnp.float32)
    nc = 48
    xcat = (xparts[:, 2 * nc:3 * nc] + xparts[:, nc:2 * nc]) + xparts[:, 0:nc]

    xs = []
    for c in range(3):
        cols = []
        for j in range(_K):
            cols.append(xcat[j * bm:(j + 1) * bm, c * 16:(c + 1) * 16])
        xs.append(jnp.concatenate(cols, axis=1))
    xx, xy, xz = xs

    qx = qt[:, 0:1]
    qy = qt[:, 1:2]
    qz = qt[:, 2:3]
    innerb = (_bf(qx) * _bf(xx) + _bf(qy) * _bf(xy)) + _bf(qz) * _bf(xz)
    hpx = 0.5 * ((xx * xx + xy * xy) + xz * xz)
    zc = innerb - hpx

    iota_c = jax.lax.broadcasted_iota(jnp.int32, (bm, _K * _GS), 1)
    ckeys = _total_order_keys(zc, 8, (_K * _GS - 1) - iota_c)
    sel_onehots = _extract_max_onehots(ckeys, _K)
    sel = sel_onehots[0]
    for oh in sel_onehots[1:]:
        sel = jnp.logical_or(sel, oh)

    dx = xx - qx
    dy = xy - qy
    dz = xz - qz
    ed = (dx * dx + dy * dy) + dz * dz

    dmin = jnp.min(jnp.where(sel, ed, _BIG), axis=1, keepdims=True)
    w = jnp.where(sel, jnp.exp((dmin - ed) * inv_sigma), 0.0)
    denom = jnp.sum(w, axis=1, keepdims=True)
    ox = jnp.sum(w * xx, axis=1, keepdims=True) / denom
    oy = jnp.sum(w * xy, axis=1, keepdims=True) / denom
    oz = jnp.sum(w * xz, axis=1, keepdims=True) / denom
    out_ref[0] = jnp.concatenate([ox, oy, oz], axis=1)


def kernel(point_cloud, query_cloud, temperature):
    b, c, n = point_cloud.shape
    _, _, m = query_cloud.shape
    bm = 256
    sigma = jnp.maximum(temperature * temperature, jnp.float32(_MIN_SIGMA))
    sigma = jnp.reshape(sigma, (1, 1)).astype(jnp.float32)

    pgf = jnp.transpose(
        jnp.reshape(point_cloud, (b, c, _GS, _NG)), (0, 3, 1, 2)
    ).reshape(b, _NG, c * _GS)
    hi = pgf.astype(jnp.bfloat16)
    r1 = pgf - hi.astype(jnp.float32)
    mid = r1.astype(jnp.bfloat16)
    lo = (r1 - mid.astype(jnp.float32)).astype(jnp.bfloat16)
    pg = jnp.concatenate([hi, mid, lo], axis=2)
    qt = jnp.transpose(query_cloud, (0, 2, 1))

    grid = (b, m // bm)
    out = pl.pallas_call(
        functools.partial(_soft_proj_block, n=n, bm=bm),
        grid=grid,
        in_specs=[
            pl.BlockSpec((1, c, n), lambda i, j: (i, 0, 0)),
            pl.BlockSpec((1, c, bm), lambda i, j: (i, 0, j)),
            pl.BlockSpec((1, bm, c), lambda i, j: (i, j, 0)),
            pl.BlockSpec((1, _NG, 3 * c * _GS), lambda i, j: (i, 0, 0)),
            pl.BlockSpec(memory_space=pltpu.SMEM),
        ],
        out_specs=pl.BlockSpec((1, bm, c), lambda i, j: (i, j, 0)),
        out_shape=jax.ShapeDtypeStruct((b, m, c), jnp.float32),
    )(point_cloud, query_cloud, qt, pg, sigma)
    return jnp.transpose(out, (0, 2, 1))


# speedup vs baseline: 99.0185x; 2.1906x over previous
import functools

import jax
import jax.numpy as jnp
from jax.experimental import pallas as pl
from jax.experimental.pallas import tpu as pltpu

_K = 16
_NG = 512
_GS = 16
_NC = _K * _GS
_MIN_SIGMA = 1e-4
_BIG = 3.0e38
_IMIN = -2**31


def _total_order_keys(vals, nbits_idx, inv_iota):
    vi = jax.lax.bitcast_convert_type(vals, jnp.int32)
    ki = jnp.where(vi < 0, vi ^ 0x7FFFFFFF, vi)
    mask = jnp.int32(-(1 << nbits_idx))
    return jnp.bitwise_or(jnp.bitwise_and(ki, mask), inv_iota)


def _extract_max_onehots(keys, steps):
    onehots = []
    for _ in range(steps):
        kmax = jnp.max(keys, axis=0, keepdims=True)
        oh = keys == kmax
        onehots.append(oh)
        keys = jnp.where(oh, _IMIN, keys)
    return onehots


def _bf(x):
    return x.astype(jnp.bfloat16).astype(jnp.float32)


def _soft_proj_block(paug_ref, qaug_ref, pg_ref, sig_ref, out_ref, *, n, bm):
    paug = paug_ref[0]
    qaug = qaug_ref[0]
    pg = pg_ref[0]
    inv_sigma = 1.0 / (sig_ref[0, 0] + 1e-8)

    zt = jax.lax.dot_general(
        paug, qaug, (((0,), (0,)), ((), ())),
        preferred_element_type=jnp.float32)

    g = zt[0:_NG, :]
    for e in range(1, _GS):
        g = jnp.maximum(g, zt[e * _NG:(e + 1) * _NG, :])

    iota_g = jax.lax.broadcasted_iota(jnp.int32, (_NG, bm), 0)
    keys = _total_order_keys(g, 9, (_NG - 1) - iota_g)
    onehots = _extract_max_onehots(keys, _K)
    ohcat = jnp.concatenate([oh.astype(jnp.bfloat16) for oh in onehots],
                            axis=1)

    xparts = jax.lax.dot_general(
        pg, ohcat, (((0,), (0,)), ((), ())),
        preferred_element_type=jnp.float32)

    xcoord = (xparts[96:144, :] + xparts[48:96, :]) + xparts[0:48, :]
    hps = xparts[144:192, :]

    def _cand(rows):
        return jnp.concatenate(
            [rows[:, j * bm:(j + 1) * bm] for j in range(_K)], axis=0)

    xx = _cand(xcoord[0:16, :])
    xy = _cand(xcoord[16:32, :])
    xz = _cand(xcoord[32:48, :])
    nh_hi = _cand(hps[0:16, :])
    nh_mid = _cand(hps[16:32, :])
    nh_lo = _cand(hps[32:48, :])

    qx = qaug[0:1, :]
    qy = qaug[1:2, :]
    qz = qaug[2:3, :]
    zc = ((_bf(qx) * _bf(xx) + _bf(qy) * _bf(xy))
          + (_bf(qz) * _bf(xz) + nh_hi)) + (nh_mid + nh_lo)

    iota_c = jax.lax.broadcasted_iota(jnp.int32, (_NC, bm), 0)
    ckeys = _total_order_keys(zc, 8, (_NC - 1) - iota_c)
    sel_onehots = _extract_max_onehots(ckeys, _K)
    sel = sel_onehots[0]
    for oh in sel_onehots[1:]:
        sel = jnp.logical_or(sel, oh)

    dx = xx - qx
    dy = xy - qy
    dz = xz - qz
    ed = (dx * dx + dy * dy) + dz * dz

    dmin = jnp.min(jnp.where(sel, ed, _BIG), axis=0, keepdims=True)
    w = jnp.where(sel, jnp.exp((dmin - ed) * inv_sigma), 0.0)
    denom = jnp.sum(w, axis=0, keepdims=True)
    ox = jnp.sum(w * xx, axis=0, keepdims=True) / denom
    oy = jnp.sum(w * xy, axis=0, keepdims=True) / denom
    oz = jnp.sum(w * xz, axis=0, keepdims=True) / denom
    out_ref[0] = jnp.concatenate([ox, oy, oz], axis=0)


def _bf16_triple(x):
    hi = x.astype(jnp.bfloat16)
    r1 = x - hi.astype(jnp.float32)
    mid = r1.astype(jnp.bfloat16)
    lo = (r1 - mid.astype(jnp.float32)).astype(jnp.bfloat16)
    return hi, mid, lo


def kernel(point_cloud, query_cloud, temperature):
    b, c, n = point_cloud.shape
    _, _, m = query_cloud.shape
    bm = 256
    sigma = jnp.maximum(temperature * temperature, jnp.float32(_MIN_SIGMA))
    sigma = jnp.reshape(sigma, (1, 1)).astype(jnp.float32)

    hp = 0.5 * jnp.sum(point_cloud * point_cloud, axis=1)
    hh, hm, hl = _bf16_triple(hp)
    paug = jnp.concatenate(
        [point_cloud,
         -hh.astype(jnp.float32)[:, None, :],
         -hm.astype(jnp.float32)[:, None, :],
         -hl.astype(jnp.float32)[:, None, :]], axis=1)
    qaug = jnp.concatenate(
        [query_cloud, jnp.ones((b, 3, m), jnp.float32)], axis=1)

    def _group(x):
        return jnp.transpose(jnp.reshape(x, (b, _GS, _NG)), (0, 2, 1))

    planes = []
    for cc in range(c):
        t = _bf16_triple(jnp.float32(1.0) * point_cloud[:, cc, :])
        planes.extend(t)
    coord_cols = [planes[0], planes[3], planes[6],
                  planes[1], planes[4], planes[7],
                  planes[2], planes[5], planes[8]]
    hp_cols = [-hh, -hm, -hl]
    pg = jnp.concatenate(
        [_group(t.astype(jnp.float32)).astype(jnp.bfloat16)
         for t in coord_cols + hp_cols], axis=2)

    grid = (b, m // bm)
    out = pl.pallas_call(
        functools.partial(_soft_proj_block, n=n, bm=bm),
        grid=grid,
        in_specs=[
            pl.BlockSpec((1, 2 * c, n), lambda i, j: (i, 0, 0)),
            pl.BlockSpec((1, 2 * c, bm), lambda i, j: (i, 0, j)),
            pl.BlockSpec((1, _NG, 4 * c * _GS), lambda i, j: (i, 0, 0)),
            pl.BlockSpec(memory_space=pltpu.SMEM),
        ],
        out_specs=pl.BlockSpec((1, c, bm), lambda i, j: (i, 0, j)),
        out_shape=jax.ShapeDtypeStruct((b, c, m), jnp.float32),
    )(paug, qaug, pg, sigma)
    return out
